# Initial kernel scaffold; baseline (speedup 1.0000x reference)
import jax
import jax.numpy as jnp
from jax.experimental import pallas as pl


def kernel(x, edges, classes, lin1_W, lin1_b, lin2_W, lin2_b, adj1_W, adj1_b, adj2_W, adj2_b):
    raise NotImplementedError("write your pallas kernel here")


# speedup vs baseline: 4.6043x; 4.6043x over previous
import functools

import jax
import jax.numpy as jnp
from jax import lax
from jax.experimental import pallas as pl
from jax.experimental.pallas import tpu as pltpu
from jax.experimental.pallas import tpu_sc as plsc

N = 10000
E = 320000
FEATS = 128
HIDDEN = 64
CLASSES = 64
DEPTH = 20

NC = 2
NS = 16
L = 16
HALF = CLASSES // NC
NPAD = 10240
RPT = NPAD // NS
CH = 128
EPT = 157 * CH
EPAD = EPT * NS
DEGW = 16

_mesh = plsc.VectorSubcoreMesh(
    core_axis_name="c", subcore_axis_name="s", num_cores=NC, num_subcores=NS)
_sc_params = pltpu.CompilerParams(use_tc_tiling_on_sc=False)


@functools.partial(
    pl.kernel,
    out_type=jax.ShapeDtypeStruct((NPAD, DEGW), jnp.float32),
    mesh=_mesh,
    compiler_params=_sc_params,
    scratch_types=[
        pltpu.VMEM_SHARED((NPAD, DEGW), jnp.float32),
        pltpu.VMEM((CH,), jnp.int32),
        pltpu.VMEM((CH, DEGW), jnp.float32),
        pltpu.VMEM((RPT, DEGW), jnp.float32),
    ],
)
def _deg_kernel(dst_hbm, deg_hbm, acc, idx, ones, buf):
    cid = lax.axis_index("c")
    sid = lax.axis_index("s")
    r0 = sid * RPT

    @pl.when(cid == 0)
    def _():
        def fill_ones(i, _):
            ones[i, pl.ds(0, L)] = jnp.ones((L,), jnp.float32)
            return 0
        lax.fori_loop(0, CH, fill_ones, 0)

        def fill_zero(i, _):
            buf[i, pl.ds(0, L)] = jnp.zeros((L,), jnp.float32)
            return 0
        lax.fori_loop(0, RPT, fill_zero, 0)
        pltpu.sync_copy(buf, acc.at[pl.ds(r0, RPT)])

    plsc.subcore_barrier()

    @pl.when(cid == 0)
    def _():
        def chunk(k, _):
            off = sid * EPT + k * CH
            pltpu.sync_copy(dst_hbm.at[pl.ds(off, CH)], idx)
            pltpu.sync_copy(ones, acc.at[idx], add=True)
            return 0
        lax.fori_loop(0, EPT // CH, chunk, 0)

    plsc.subcore_barrier()

    @pl.when(cid == 0)
    def _():
        pltpu.sync_copy(acc.at[pl.ds(r0, RPT)], buf)
        pltpu.sync_copy(buf, deg_hbm.at[pl.ds(r0, RPT)])


@functools.partial(
    pl.kernel,
    out_type=jax.ShapeDtypeStruct((NC * NPAD, HALF), jnp.float32),
    mesh=_mesh,
    compiler_params=_sc_params,
    scratch_types=[
        pltpu.VMEM_SHARED((NPAD, HALF), jnp.float32),
        pltpu.VMEM((CH,), jnp.int32),
        pltpu.VMEM((CH,), jnp.int32),
        pltpu.VMEM((CH, HALF), jnp.float32),
        pltpu.VMEM((RPT, HALF), jnp.float32),
        pltpu.VMEM((RPT, HALF), jnp.float32),
        pltpu.VMEM((RPT, HALF), jnp.float32),
        pltpu.VMEM((RPT, HALF), jnp.float32),
    ],
)
def _diff_kernel(u0_hbm, d2_hbm, gsrc_hbm, dst_hbm, u_hbm,
                 acc, gi, di, rows, u0b, d2b, accb, zb):
    cid = lax.axis_index("c")
    sid = lax.axis_index("s")
    r0 = sid * RPT
    g0 = cid * NPAD + r0

    pltpu.sync_copy(u0_hbm.at[pl.ds(g0, RPT)], u0b)
    pltpu.sync_copy(u0b, u_hbm.at[pl.ds(g0, RPT)])
    pltpu.sync_copy(d2_hbm.at[pl.ds(r0, RPT)], d2b)

    def prep(r, _):
        for colo in (0, L):
            u0b[r, pl.ds(colo, L)] = u0b[r, pl.ds(colo, L)] * 0.1
            zb[r, pl.ds(colo, L)] = jnp.zeros((L,), jnp.float32)
        return 0
    lax.fori_loop(0, RPT, prep, 0)
    pltpu.sync_copy(zb, acc.at[pl.ds(r0, RPT)])
    plsc.subcore_barrier()

    def step(t, _):
        def chunk(k, _):
            off = sid * EPT + k * CH
            pltpu.sync_copy(gsrc_hbm.at[cid, pl.ds(off, CH)], gi)
            pltpu.sync_copy(dst_hbm.at[pl.ds(off, CH)], di)
            pltpu.sync_copy(u_hbm.at[gi], rows)
            pltpu.sync_copy(rows, acc.at[di], add=True)
            return 0
        lax.fori_loop(0, EPT // CH, chunk, 0)
        plsc.subcore_barrier()

        pltpu.sync_copy(acc.at[pl.ds(r0, RPT)], accb)
        pltpu.sync_copy(zb, acc.at[pl.ds(r0, RPT)])

        def upd(r, _):
            for colo in (0, L):
                accb[r, pl.ds(colo, L)] = (
                    accb[r, pl.ds(colo, L)] * d2b[r, pl.ds(colo, L)]
                    + u0b[r, pl.ds(colo, L)])
            return 0
        lax.fori_loop(0, RPT, upd, 0)
        pltpu.sync_copy(accb, u_hbm.at[pl.ds(g0, RPT)])
        plsc.subcore_barrier()
        return 0
    lax.fori_loop(0, DEPTH, step, 0)


_BN = 512


def _prep_body(x_ref, w1_ref, b1_ref, w2_ref, b2_ref, deg_ref, u0_ref, d2_ref):
    xb = x_ref[...]
    h1 = jnp.maximum(
        jnp.dot(xb, w1_ref[...], preferred_element_type=jnp.float32)
        + b1_ref[...], 0.0)
    h0 = jnp.dot(h1, w2_ref[...], preferred_element_type=jnp.float32) + b2_ref[...]
    deg = jnp.maximum(deg_ref[...], 1.0)
    dis = lax.rsqrt(deg)
    u0 = h0 * dis
    u0_ref[0, :, :] = u0[:, :HALF]
    u0_ref[1, :, :] = u0[:, HALF:]
    d2_ref[...] = jnp.broadcast_to(0.9 / deg, (_BN, HALF))


def _prep_call(x_pad, w1, b1, w2, b2, deg2d):
    nb = NPAD // _BN
    full = lambda shape: pl.BlockSpec(shape, lambda i: (0,) * len(shape))
    return pl.pallas_call(
        _prep_body,
        grid=(nb,),
        in_specs=[
            pl.BlockSpec((_BN, FEATS), lambda i: (i, 0)),
            full((FEATS, HIDDEN)),
            full((1, HIDDEN)),
            full((HIDDEN, CLASSES)),
            full((1, CLASSES)),
            pl.BlockSpec((_BN, 1), lambda i: (i, 0)),
        ],
        out_specs=[
            pl.BlockSpec((NC, _BN, HALF), lambda i: (0, i, 0)),
            pl.BlockSpec((_BN, HALF), lambda i: (i, 0)),
        ],
        out_shape=[
            jax.ShapeDtypeStruct((NC, NPAD, HALF), jnp.float32),
            jax.ShapeDtypeStruct((NPAD, HALF), jnp.float32),
        ],
    )(x_pad, w1, b1, w2, b2, deg2d)


_BD = 256
_CH2 = CLASSES * HIDDEN


def _adj_body(u20_ref, deg_ref, x_ref, w1x_ref, r_ref, b1f_ref, w2f_ref,
              s_ref, b2_ref, uo_ref):
    deg = jnp.maximum(deg_ref[...], 1.0)
    dis = lax.rsqrt(deg)
    sqd = deg * dis
    h20 = jnp.concatenate([u20_ref[0], u20_ref[1]], axis=1) * sqd
    pre = jnp.dot(x_ref[...], w1x_ref[...], preferred_element_type=jnp.float32)
    pre = pre + jnp.dot(h20, r_ref[...], preferred_element_type=jnp.float32)
    pre = pre + b1f_ref[...]
    z = jnp.maximum(pre, 0.0) * w2f_ref[...]
    o = jnp.dot(z, s_ref[...], preferred_element_type=jnp.float32) + b2_ref[...]
    uo = o * dis
    uo_ref[0, :, :] = uo[:, :HALF]
    uo_ref[1, :, :] = uo[:, HALF:]


def _adj_call(u20, deg2d, x_pad, w1x, rmat, b1f, w2f, smat, b2r):
    nb = NPAD // _BD
    full = lambda shape: pl.BlockSpec(shape, lambda i: (0,) * len(shape))
    return pl.pallas_call(
        _adj_body,
        grid=(nb,),
        in_specs=[
            pl.BlockSpec((NC, _BD, HALF), lambda i: (0, i, 0)),
            pl.BlockSpec((_BD, 1), lambda i: (i, 0)),
            pl.BlockSpec((_BD, FEATS), lambda i: (i, 0)),
            full((FEATS, _CH2)),
            full((CLASSES, _CH2)),
            full((1, _CH2)),
            full((1, _CH2)),
            full((_CH2, CLASSES)),
            full((1, CLASSES)),
        ],
        out_specs=pl.BlockSpec((NC, _BD, HALF), lambda i: (0, i, 0)),
        out_shape=jax.ShapeDtypeStruct((NC, NPAD, HALF), jnp.float32),
    )(u20, deg2d, x_pad, w1x, rmat, b1f, w2f, smat, b2r)


def _fin_body(u_ref, deg_ref, o_ref):
    deg = jnp.maximum(deg_ref[...], 1.0)
    sqd = deg * lax.rsqrt(deg)
    o_ref[...] = jnp.concatenate([u_ref[0], u_ref[1]], axis=1) * sqd


def _fin_call(u40, deg2d):
    nb = NPAD // _BN
    return pl.pallas_call(
        _fin_body,
        grid=(nb,),
        in_specs=[
            pl.BlockSpec((NC, _BN, HALF), lambda i: (0, i, 0)),
            pl.BlockSpec((_BN, 1), lambda i: (i, 0)),
        ],
        out_specs=pl.BlockSpec((_BN, CLASSES), lambda i: (i, 0)),
        out_shape=jax.ShapeDtypeStruct((NPAD, CLASSES), jnp.float32),
    )(u40, deg2d)


def kernel(x, edges, classes, lin1_W, lin1_b, lin2_W, lin2_b,
           adj1_W, adj1_b, adj2_W, adj2_b):
    src = edges[0].astype(jnp.int32)
    dst = edges[1].astype(jnp.int32)

    pad = EPAD - E
    pad_i = jnp.arange(pad, dtype=jnp.int32)
    srcp = jnp.concatenate([src, (pad_i * 61) % N])
    dstp = jnp.concatenate([dst, N + (pad_i % NS)])
    gsrc = jnp.stack([srcp, srcp + NPAD])

    x_pad = jnp.zeros((NPAD, FEATS), jnp.float32).at[:N].set(x)

    degw = _deg_kernel(dstp)
    deg2d = degw[:, :1]

    w1x = adj1_W[:, 1:, :].transpose(1, 0, 2).reshape(FEATS, _CH2)
    rmat = (jnp.eye(CLASSES, dtype=jnp.float32)[:, :, None]
            * adj1_W[:, 0, :][None, :, :]).reshape(CLASSES, _CH2)
    b1f = adj1_b.reshape(1, _CH2)
    w2f = adj2_W[:, :, 0].reshape(1, _CH2)
    smat = jnp.repeat(jnp.eye(CLASSES, dtype=jnp.float32), HIDDEN, axis=0)
    b2r = adj2_b[:, 0].reshape(1, CLASSES)

    u0, d2 = _prep_call(x_pad, lin1_W, lin1_b.reshape(1, HIDDEN),
                        lin2_W, lin2_b.reshape(1, CLASSES), deg2d)

    u20 = _diff_kernel(u0.reshape(NC * NPAD, HALF), d2, gsrc, dstp)

    u0p = _adj_call(u20.reshape(NC, NPAD, HALF), deg2d, x_pad,
                    w1x, rmat, b1f, w2f, smat, b2r)

    u40 = _diff_kernel(u0p.reshape(NC * NPAD, HALF), d2, gsrc, dstp)

    h = _fin_call(u40.reshape(NC, NPAD, HALF), deg2d)
    return h[:N]


# speedup vs baseline: 20.9939x; 4.5596x over previous
import functools

import jax
import jax.numpy as jnp
from jax import lax
from jax.experimental import pallas as pl
from jax.experimental.pallas import tpu as pltpu
from jax.experimental.pallas import tpu_sc as plsc

N = 10000
E = 320000
FEATS = 128
HIDDEN = 64
CLASSES = 64
DEPTH = 20

NC = 2
NS = 16
L = 16
HALF = CLASSES // NC
NPAD = 10240
RPT = NPAD // NS
CH = 128
NBUF = 5
CHK = 160
EPT = CHK * CH
EPAD = EPT * NS
DEGW = 16
DCHK = EPAD // (NC * NS * CH)

_mesh = plsc.VectorSubcoreMesh(
    core_axis_name="c", subcore_axis_name="s", num_cores=NC, num_subcores=NS)
_sc_params = pltpu.CompilerParams(use_tc_tiling_on_sc=False)


@functools.partial(
    pl.kernel,
    out_type=jax.ShapeDtypeStruct((NC, NPAD, DEGW), jnp.float32),
    mesh=_mesh,
    compiler_params=_sc_params,
    scratch_types=(
        [
            pltpu.VMEM_SHARED((NPAD, DEGW), jnp.float32),
            pltpu.VMEM((DCHK, CH), jnp.int32),
            pltpu.VMEM((CH, DEGW), jnp.float32),
            pltpu.VMEM((RPT, DEGW), jnp.float32),
        ]
        + [pltpu.SemaphoreType.DMA] * NBUF
    ),
)
def _deg_kernel(dst_hbm, deg_hbm, acc, didx, ones, buf, *ss):
    cid = lax.axis_index("c")
    sid = lax.axis_index("s")
    r0 = sid * RPT

    pltpu.sync_copy(dst_hbm.at[cid * NS + sid], didx)

    def fill_ones(i, _):
        ones[i, pl.ds(0, L)] = jnp.ones((L,), jnp.float32)
        return 0
    lax.fori_loop(0, CH, fill_ones, 0)

    def fill_zero(i, _):
        buf[i, pl.ds(0, L)] = jnp.zeros((L,), jnp.float32)
        return 0
    lax.fori_loop(0, RPT, fill_zero, 0)
    pltpu.sync_copy(buf, acc.at[pl.ds(r0, RPT)])
    plsc.subcore_barrier()

    def round_body(r, _):
        for b in range(NBUF):
            k = r * NBUF + b

            @pl.when(r > 0)
            def _(b=b, k=k):
                pltpu.make_async_copy(ones, acc.at[didx.at[k]], ss[b]).wait()

            pltpu.async_copy(ones, acc.at[didx.at[k]], ss[b], add=True)
        return 0
    lax.fori_loop(0, DCHK // NBUF, round_body, 0)
    for b in range(NBUF):
        pltpu.make_async_copy(
            ones, acc.at[didx.at[DCHK - NBUF + b]], ss[b]).wait()

    plsc.subcore_barrier()
    pltpu.sync_copy(acc.at[pl.ds(r0, RPT)], buf)
    pltpu.sync_copy(buf, deg_hbm.at[cid, pl.ds(r0, RPT)])


@functools.partial(
    pl.kernel,
    out_type=jax.ShapeDtypeStruct((NC * NPAD, HALF), jnp.float32),
    mesh=_mesh,
    compiler_params=_sc_params,
    scratch_types=(
        [
            pltpu.VMEM_SHARED((NPAD, HALF), jnp.float32),
            pltpu.VMEM((NBUF, CH), jnp.int32),
            pltpu.VMEM((CHK, CH), jnp.int32),
            pltpu.VMEM((NBUF, CH, HALF), jnp.float32),
            pltpu.VMEM((RPT, HALF), jnp.float32),
            pltpu.VMEM((RPT, HALF), jnp.float32),
            pltpu.VMEM((RPT, HALF), jnp.float32),
            pltpu.VMEM((CH, HALF), jnp.float32),
        ]
        + [pltpu.SemaphoreType.DMA] * (3 * NBUF)
    ),
)
def _diff_kernel(u0_hbm, d2_hbm, gsrc_hbm, dst_hbm, u_hbm,
                 acc, gidx, didx, rows, u0b, d2b, accb, zbs, *sems):
    sg = sems[:NBUF]
    ss = sems[NBUF:2 * NBUF]
    si = sems[2 * NBUF:]
    cid = lax.axis_index("c")
    sid = lax.axis_index("s")
    tid = cid * NS + sid
    r0 = sid * RPT
    g0 = cid * NPAD + r0

    pltpu.sync_copy(dst_hbm.at[sid], didx)
    pltpu.sync_copy(u0_hbm.at[pl.ds(g0, RPT)], u0b)
    pltpu.sync_copy(u0b, u_hbm.at[pl.ds(g0, RPT)])
    pltpu.sync_copy(d2_hbm.at[pl.ds(r0, RPT)], d2b)

    def prep(r, _):
        for colo in (0, L):
            u0b[r, pl.ds(colo, L)] = u0b[r, pl.ds(colo, L)] * 0.1
        return 0
    lax.fori_loop(0, RPT, prep, 0)

    def zfill(i, _):
        for colo in (0, L):
            zbs[i, pl.ds(colo, L)] = jnp.zeros((L,), jnp.float32)
        return 0
    lax.fori_loop(0, CH, zfill, 0)
    for z in range(RPT // CH):
        pltpu.sync_copy(zbs, acc.at[pl.ds(r0 + z * CH, CH)])
    plsc.subcore_barrier()

    def step(t, _):
        for b in range(NBUF):
            pltpu.async_copy(gsrc_hbm.at[tid, b], gidx.at[b], si[b])

        def round_body(r, _):
            descs = []
            for b in range(NBUF):
                k = r * NBUF + b

                @pl.when(r > 0)
                def _(b=b, k=k):
                    pltpu.make_async_copy(
                        rows.at[b], acc.at[didx.at[k]], ss[b]).wait()

                pltpu.make_async_copy(
                    gsrc_hbm.at[tid, 0], gidx.at[b], si[b]).wait()
                descs.append(
                    pltpu.async_copy(u_hbm.at[gidx.at[b]], rows.at[b], sg[b]))
            for b in range(NBUF):
                k = r * NBUF + b
                descs[b].wait()

                @pl.when(k + NBUF < CHK)
                def _(b=b, k=k):
                    pltpu.async_copy(
                        gsrc_hbm.at[tid, k + NBUF], gidx.at[b], si[b])

                pltpu.async_copy(rows.at[b], acc.at[didx.at[k]], ss[b],
                                 add=True)
            return 0
        lax.fori_loop(0, CHK // NBUF, round_body, 0)
        for b in range(NBUF):
            pltpu.make_async_copy(
                rows.at[b], acc.at[didx.at[CHK - NBUF + b]], ss[b]).wait()
        plsc.subcore_barrier()

        pltpu.sync_copy(acc.at[pl.ds(r0, RPT)], accb)
        for z in range(RPT // CH):
            pltpu.sync_copy(zbs, acc.at[pl.ds(r0 + z * CH, CH)])

        def upd(r, _):
            for colo in (0, L):
                accb[r, pl.ds(colo, L)] = (
                    accb[r, pl.ds(colo, L)] * d2b[r, pl.ds(colo, L)]
                    + u0b[r, pl.ds(colo, L)])
            return 0
        lax.fori_loop(0, RPT, upd, 0)
        pltpu.sync_copy(accb, u_hbm.at[pl.ds(g0, RPT)])
        plsc.subcore_barrier()
        return 0
    lax.fori_loop(0, DEPTH, step, 0)


_BN = 512


def _prep_body(x_ref, w1_ref, b1_ref, w2_ref, b2_ref, deg_ref, u0_ref, d2_ref):
    xb = x_ref[...]
    h1 = jnp.maximum(
        jnp.dot(xb, w1_ref[...], preferred_element_type=jnp.float32)
        + b1_ref[...], 0.0)
    h0 = jnp.dot(h1, w2_ref[...], preferred_element_type=jnp.float32) + b2_ref[...]
    deg = jnp.maximum(deg_ref[0, :, :1] + deg_ref[1, :, :1], 1.0)
    dis = lax.rsqrt(deg)
    u0 = h0 * dis
    u0_ref[0, :, :] = u0[:, :HALF]
    u0_ref[1, :, :] = u0[:, HALF:]
    d2_ref[...] = jnp.broadcast_to(0.9 / deg, (_BN, HALF))


def _prep_call(x_pad, w1, b1, w2, b2, degw):
    nb = NPAD // _BN
    full = lambda shape: pl.BlockSpec(shape, lambda i: (0,) * len(shape))
    return pl.pallas_call(
        _prep_body,
        grid=(nb,),
        in_specs=[
            pl.BlockSpec((_BN, FEATS), lambda i: (i, 0)),
            full((FEATS, HIDDEN)),
            full((1, HIDDEN)),
            full((HIDDEN, CLASSES)),
            full((1, CLASSES)),
            pl.BlockSpec((NC, _BN, DEGW), lambda i: (0, i, 0)),
        ],
        out_specs=[
            pl.BlockSpec((NC, _BN, HALF), lambda i: (0, i, 0)),
            pl.BlockSpec((_BN, HALF), lambda i: (i, 0)),
        ],
        out_shape=[
            jax.ShapeDtypeStruct((NC, NPAD, HALF), jnp.float32),
            jax.ShapeDtypeStruct((NPAD, HALF), jnp.float32),
        ],
    )(x_pad, w1, b1, w2, b2, degw)


_BD = 256
_CH2 = CLASSES * HIDDEN


def _adj_body(u20_ref, deg_ref, x_ref, w1x_ref, r_ref, b1f_ref, w2f_ref,
              s_ref, b2_ref, uo_ref):
    deg = jnp.maximum(deg_ref[0, :, :1] + deg_ref[1, :, :1], 1.0)
    dis = lax.rsqrt(deg)
    sqd = deg * dis
    h20 = jnp.concatenate([u20_ref[0], u20_ref[1]], axis=1) * sqd
    pre = jnp.dot(x_ref[...], w1x_ref[...], preferred_element_type=jnp.float32)
    pre = pre + jnp.dot(h20, r_ref[...], preferred_element_type=jnp.float32)
    pre = pre + b1f_ref[...]
    z = jnp.maximum(pre, 0.0) * w2f_ref[...]
    o = jnp.dot(z, s_ref[...], preferred_element_type=jnp.float32) + b2_ref[...]
    uo = o * dis
    uo_ref[0, :, :] = uo[:, :HALF]
    uo_ref[1, :, :] = uo[:, HALF:]


def _adj_call(u20, degw, x_pad, w1x, rmat, b1f, w2f, smat, b2r):
    nb = NPAD // _BD
    full = lambda shape: pl.BlockSpec(shape, lambda i: (0,) * len(shape))
    return pl.pallas_call(
        _adj_body,
        grid=(nb,),
        in_specs=[
            pl.BlockSpec((NC, _BD, HALF), lambda i: (0, i, 0)),
            pl.BlockSpec((NC, _BD, DEGW), lambda i: (0, i, 0)),
            pl.BlockSpec((_BD, FEATS), lambda i: (i, 0)),
            full((FEATS, _CH2)),
            full((CLASSES, _CH2)),
            full((1, _CH2)),
            full((1, _CH2)),
            full((_CH2, CLASSES)),
            full((1, CLASSES)),
        ],
        out_specs=pl.BlockSpec((NC, _BD, HALF), lambda i: (0, i, 0)),
        out_shape=jax.ShapeDtypeStruct((NC, NPAD, HALF), jnp.float32),
    )(u20, degw, x_pad, w1x, rmat, b1f, w2f, smat, b2r)


def _fin_body(u_ref, deg_ref, o_ref):
    deg = jnp.maximum(deg_ref[0, :, :1] + deg_ref[1, :, :1], 1.0)
    sqd = deg * lax.rsqrt(deg)
    o_ref[...] = jnp.concatenate([u_ref[0], u_ref[1]], axis=1) * sqd


def _fin_call(u40, degw):
    nb = NPAD // _BN
    return pl.pallas_call(
        _fin_body,
        grid=(nb,),
        in_specs=[
            pl.BlockSpec((NC, _BN, HALF), lambda i: (0, i, 0)),
            pl.BlockSpec((NC, _BN, DEGW), lambda i: (0, i, 0)),
        ],
        out_specs=pl.BlockSpec((_BN, CLASSES), lambda i: (i, 0)),
        out_shape=jax.ShapeDtypeStruct((NPAD, CLASSES), jnp.float32),
    )(u40, degw)


def kernel(x, edges, classes, lin1_W, lin1_b, lin2_W, lin2_b,
           adj1_W, adj1_b, adj2_W, adj2_b):
    src = edges[0].astype(jnp.int32)
    dst = edges[1].astype(jnp.int32)

    pad = EPAD - E
    pad_i = jnp.arange(pad, dtype=jnp.int32)
    srcp = jnp.concatenate([src, (pad_i * 61) % N])
    dstp = jnp.concatenate([dst, N + (pad_i % NS)])
    gsrc = jnp.stack([srcp, srcp + NPAD]).reshape(NC * NS, CHK, CH)
    dst3 = dstp.reshape(NS, CHK, CH)
    dst32 = dstp.reshape(NC * NS, DCHK, CH)

    x_pad = jnp.zeros((NPAD, FEATS), jnp.float32).at[:N].set(x)

    degw = _deg_kernel(dst32)

    w1x = adj1_W[:, 1:, :].transpose(1, 0, 2).reshape(FEATS, _CH2)
    rmat = (jnp.eye(CLASSES, dtype=jnp.float32)[:, :, None]
            * adj1_W[:, 0, :][None, :, :]).reshape(CLASSES, _CH2)
    b1f = adj1_b.reshape(1, _CH2)
    w2f = adj2_W[:, :, 0].reshape(1, _CH2)
    smat = jnp.repeat(jnp.eye(CLASSES, dtype=jnp.float32), HIDDEN, axis=0)
    b2r = adj2_b[:, 0].reshape(1, CLASSES)

    u0, d2 = _prep_call(x_pad, lin1_W, lin1_b.reshape(1, HIDDEN),
                        lin2_W, lin2_b.reshape(1, CLASSES), degw)

    u20 = _diff_kernel(u0.reshape(NC * NPAD, HALF), d2, gsrc, dst3)

    u0p = _adj_call(u20.reshape(NC, NPAD, HALF), degw, x_pad,
                    w1x, rmat, b1f, w2f, smat, b2r)

    u40 = _diff_kernel(u0p.reshape(NC * NPAD, HALF), d2, gsrc, dst3)

    h = _fin_call(u40.reshape(NC, NPAD, HALF), degw)
    return h[:N]


# speedup vs baseline: 23.7889x; 1.1331x over previous
import functools

import jax
import jax.numpy as jnp
from jax import lax
from jax.experimental import pallas as pl
from jax.experimental.pallas import tpu as pltpu
from jax.experimental.pallas import tpu_sc as plsc

N = 10000
E = 320000
FEATS = 128
HIDDEN = 64
CLASSES = 64
DEPTH = 20

NC = 2
NS = 16
L = 16
HALF = CLASSES // NC
NPAD = 10240
RPT = NPAD // NS
CH = 128
NBUF = 8
CHK = 160
EPT = CHK * CH
EPAD = EPT * NS
DEGW = 16
DCHK = EPAD // (NC * NS * CH)

_mesh = plsc.VectorSubcoreMesh(
    core_axis_name="c", subcore_axis_name="s", num_cores=NC, num_subcores=NS)
_sc_params = pltpu.CompilerParams(use_tc_tiling_on_sc=False)


@functools.partial(
    pl.kernel,
    out_type=jax.ShapeDtypeStruct((NC, NPAD, DEGW), jnp.float32),
    mesh=_mesh,
    compiler_params=_sc_params,
    scratch_types=(
        [
            pltpu.VMEM_SHARED((NPAD, DEGW), jnp.float32),
            pltpu.VMEM((DCHK, CH), jnp.int32),
            pltpu.VMEM((CH, DEGW), jnp.float32),
            pltpu.VMEM((RPT, DEGW), jnp.float32),
        ]
        + [pltpu.SemaphoreType.DMA] * NBUF
    ),
)
def _deg_kernel(dst_hbm, deg_hbm, acc, didx, ones, buf, *ss):
    cid = lax.axis_index("c")
    sid = lax.axis_index("s")
    r0 = sid * RPT

    pltpu.sync_copy(dst_hbm.at[cid * NS + sid], didx)

    def fill_ones(i, _):
        ones[i, pl.ds(0, L)] = jnp.ones((L,), jnp.float32)
        return 0
    lax.fori_loop(0, CH, fill_ones, 0)

    def fill_zero(i, _):
        buf[i, pl.ds(0, L)] = jnp.zeros((L,), jnp.float32)
        return 0
    lax.fori_loop(0, RPT, fill_zero, 0)
    pltpu.sync_copy(buf, acc.at[pl.ds(r0, RPT)])
    plsc.subcore_barrier()

    def round_body(r, _):
        for b in range(NBUF):
            k = r * NBUF + b

            @pl.when(r > 0)
            def _(b=b, k=k):
                pltpu.make_async_copy(ones, acc.at[didx.at[k]], ss[b]).wait()

            pltpu.async_copy(ones, acc.at[didx.at[k]], ss[b], add=True)
        return 0
    lax.fori_loop(0, DCHK // NBUF, round_body, 0)
    for b in range(NBUF):
        pltpu.make_async_copy(
            ones, acc.at[didx.at[DCHK - NBUF + b]], ss[b]).wait()

    plsc.subcore_barrier()
    pltpu.sync_copy(acc.at[pl.ds(r0, RPT)], buf)
    pltpu.sync_copy(buf, deg_hbm.at[cid, pl.ds(r0, RPT)])


@functools.partial(
    pl.kernel,
    out_type=jax.ShapeDtypeStruct((NC * NPAD, HALF), jnp.float32),
    mesh=_mesh,
    compiler_params=_sc_params,
    scratch_types=(
        [
            pltpu.VMEM_SHARED((NPAD, HALF), jnp.float32),
            pltpu.VMEM((NBUF, 2, CH), jnp.int32),
            pltpu.VMEM((NBUF, CH, HALF), jnp.float32),
            pltpu.VMEM((RPT, HALF), jnp.float32),
            pltpu.VMEM((RPT, HALF), jnp.float32),
            pltpu.VMEM((RPT, HALF), jnp.float32),
            pltpu.VMEM((CH, HALF), jnp.float32),
        ]
        + [pltpu.SemaphoreType.DMA] * (3 * NBUF)
    ),
)
def _diff_kernel(u0_hbm, d2_hbm, iidx_hbm, u_hbm,
                 acc, iring, rows, u0b, d2b, accb, zbs, *sems):
    sg = sems[:NBUF]
    ss = sems[NBUF:2 * NBUF]
    si = sems[2 * NBUF:]
    cid = lax.axis_index("c")
    sid = lax.axis_index("s")
    tid = cid * NS + sid
    r0 = sid * RPT
    g0 = cid * NPAD + r0
    ZB = RPT // CH

    pltpu.sync_copy(u0_hbm.at[pl.ds(g0, RPT)], u0b)
    pltpu.sync_copy(u0b, u_hbm.at[pl.ds(g0, RPT)])
    pltpu.sync_copy(d2_hbm.at[pl.ds(r0, RPT)], d2b)

    def prep(r, _):
        for colo in (0, L):
            u0b[r, pl.ds(colo, L)] = u0b[r, pl.ds(colo, L)] * 0.1
        return 0
    lax.fori_loop(0, RPT, prep, 0)

    def zfill(i, _):
        for colo in (0, L):
            zbs[i, pl.ds(colo, L)] = jnp.zeros((L,), jnp.float32)
        return 0
    lax.fori_loop(0, CH, zfill, 0)
    for z in range(ZB):
        pltpu.sync_copy(zbs, acc.at[pl.ds(r0 + z * CH, CH)])
    plsc.subcore_barrier()

    def step(t, _):
        for b in range(NBUF):
            pltpu.async_copy(iidx_hbm.at[tid, b], iring.at[b], si[b])

        def round_body(r, _):
            descs = []
            for b in range(NBUF):
                k = r * NBUF + b

                @pl.when(r > 0)
                def _(b=b, k=k):
                    pltpu.make_async_copy(
                        rows.at[b], acc.at[iring.at[b, 1]], ss[b]).wait()

                pltpu.make_async_copy(
                    iidx_hbm.at[tid, 0], iring.at[b], si[b]).wait()
                descs.append(
                    pltpu.async_copy(
                        u_hbm.at[iring.at[b, 0]], rows.at[b], sg[b]))
            for b in range(NBUF):
                k = r * NBUF + b
                descs[b].wait()
                pltpu.async_copy(rows.at[b], acc.at[iring.at[b, 1]], ss[b],
                                 add=True)

                @pl.when(k + NBUF < CHK)
                def _(b=b, k=k):
                    pltpu.async_copy(
                        iidx_hbm.at[tid, k + NBUF], iring.at[b], si[b])
            return 0
        lax.fori_loop(0, CHK // NBUF, round_body, 0)
        for b in range(NBUF):
            pltpu.make_async_copy(
                rows.at[b], acc.at[iring.at[b, 1]], ss[b]).wait()
        plsc.subcore_barrier()

        for z in range(ZB):
            pltpu.async_copy(acc.at[pl.ds(r0 + z * CH, CH)],
                             accb.at[pl.ds(z * CH, CH)], sg[z])
        for z in range(ZB):
            pltpu.make_async_copy(acc.at[pl.ds(r0 + z * CH, CH)],
                                  accb.at[pl.ds(z * CH, CH)], sg[z]).wait()
            pltpu.async_copy(zbs, acc.at[pl.ds(r0 + z * CH, CH)], ss[z])

            def upd(r, _):
                for colo in (0, L):
                    accb[r, pl.ds(colo, L)] = (
                        accb[r, pl.ds(colo, L)] * d2b[r, pl.ds(colo, L)]
                        + u0b[r, pl.ds(colo, L)])
                return 0
            lax.fori_loop(z * CH, (z + 1) * CH, upd, 0)
            pltpu.async_copy(accb.at[pl.ds(z * CH, CH)],
                             u_hbm.at[pl.ds(g0 + z * CH, CH)], si[z])
        for z in range(ZB):
            pltpu.make_async_copy(zbs, acc.at[pl.ds(r0 + z * CH, CH)],
                                  ss[z]).wait()
            pltpu.make_async_copy(accb.at[pl.ds(z * CH, CH)],
                                  u_hbm.at[pl.ds(g0 + z * CH, CH)],
                                  si[z]).wait()
        plsc.subcore_barrier()
        return 0
    lax.fori_loop(0, DEPTH, step, 0)


_BN = 512


def _prep_body(x_ref, w1_ref, b1_ref, w2_ref, b2_ref, deg_ref, u0_ref, d2_ref):
    xb = x_ref[...]
    h1 = jnp.maximum(
        jnp.dot(xb, w1_ref[...], preferred_element_type=jnp.float32)
        + b1_ref[...], 0.0)
    h0 = jnp.dot(h1, w2_ref[...], preferred_element_type=jnp.float32) + b2_ref[...]
    deg = jnp.maximum(deg_ref[0, :, :1] + deg_ref[1, :, :1], 1.0)
    dis = lax.rsqrt(deg)
    u0 = h0 * dis
    u0_ref[0, :, :] = u0[:, :HALF]
    u0_ref[1, :, :] = u0[:, HALF:]
    d2_ref[...] = jnp.broadcast_to(0.9 / deg, (_BN, HALF))


def _prep_call(x_pad, w1, b1, w2, b2, degw):
    nb = NPAD // _BN
    full = lambda shape: pl.BlockSpec(shape, lambda i: (0,) * len(shape))
    return pl.pallas_call(
        _prep_body,
        grid=(nb,),
        in_specs=[
            pl.BlockSpec((_BN, FEATS), lambda i: (i, 0)),
            full((FEATS, HIDDEN)),
            full((1, HIDDEN)),
            full((HIDDEN, CLASSES)),
            full((1, CLASSES)),
            pl.BlockSpec((NC, _BN, DEGW), lambda i: (0, i, 0)),
        ],
        out_specs=[
            pl.BlockSpec((NC, _BN, HALF), lambda i: (0, i, 0)),
            pl.BlockSpec((_BN, HALF), lambda i: (i, 0)),
        ],
        out_shape=[
            jax.ShapeDtypeStruct((NC, NPAD, HALF), jnp.float32),
            jax.ShapeDtypeStruct((NPAD, HALF), jnp.float32),
        ],
    )(x_pad, w1, b1, w2, b2, degw)


_BD = 256
_CH2 = CLASSES * HIDDEN


def _adj_body(u20_ref, deg_ref, x_ref, w1x_ref, r_ref, b1f_ref, w2f_ref,
              s_ref, b2_ref, uo_ref):
    deg = jnp.maximum(deg_ref[0, :, :1] + deg_ref[1, :, :1], 1.0)
    dis = lax.rsqrt(deg)
    sqd = deg * dis
    h20 = jnp.concatenate([u20_ref[0], u20_ref[1]], axis=1) * sqd
    pre = jnp.dot(x_ref[...], w1x_ref[...], preferred_element_type=jnp.float32)
    pre = pre + jnp.dot(h20, r_ref[...], preferred_element_type=jnp.float32)
    pre = pre + b1f_ref[...]
    z = jnp.maximum(pre, 0.0) * w2f_ref[...]
    o = jnp.dot(z, s_ref[...], preferred_element_type=jnp.float32) + b2_ref[...]
    uo = o * dis
    uo_ref[0, :, :] = uo[:, :HALF]
    uo_ref[1, :, :] = uo[:, HALF:]


def _adj_call(u20, degw, x_pad, w1x, rmat, b1f, w2f, smat, b2r):
    nb = NPAD // _BD
    full = lambda shape: pl.BlockSpec(shape, lambda i: (0,) * len(shape))
    return pl.pallas_call(
        _adj_body,
        grid=(nb,),
        in_specs=[
            pl.BlockSpec((NC, _BD, HALF), lambda i: (0, i, 0)),
            pl.BlockSpec((NC, _BD, DEGW), lambda i: (0, i, 0)),
            pl.BlockSpec((_BD, FEATS), lambda i: (i, 0)),
            full((FEATS, _CH2)),
            full((CLASSES, _CH2)),
            full((1, _CH2)),
            full((1, _CH2)),
            full((_CH2, CLASSES)),
            full((1, CLASSES)),
        ],
        out_specs=pl.BlockSpec((NC, _BD, HALF), lambda i: (0, i, 0)),
        out_shape=jax.ShapeDtypeStruct((NC, NPAD, HALF), jnp.float32),
    )(u20, degw, x_pad, w1x, rmat, b1f, w2f, smat, b2r)


def _fin_body(u_ref, deg_ref, o_ref):
    deg = jnp.maximum(deg_ref[0, :, :1] + deg_ref[1, :, :1], 1.0)
    sqd = deg * lax.rsqrt(deg)
    o_ref[...] = jnp.concatenate([u_ref[0], u_ref[1]], axis=1) * sqd


def _fin_call(u40, degw):
    nb = NPAD // _BN
    return pl.pallas_call(
        _fin_body,
        grid=(nb,),
        in_specs=[
            pl.BlockSpec((NC, _BN, HALF), lambda i: (0, i, 0)),
            pl.BlockSpec((NC, _BN, DEGW), lambda i: (0, i, 0)),
        ],
        out_specs=pl.BlockSpec((_BN, CLASSES), lambda i: (i, 0)),
        out_shape=jax.ShapeDtypeStruct((NPAD, CLASSES), jnp.float32),
    )(u40, degw)


def kernel(x, edges, classes, lin1_W, lin1_b, lin2_W, lin2_b,
           adj1_W, adj1_b, adj2_W, adj2_b):
    src = edges[0].astype(jnp.int32)
    dst = edges[1].astype(jnp.int32)

    pad = EPAD - E
    pad_i = jnp.arange(pad, dtype=jnp.int32)
    srcp = jnp.concatenate([src, (pad_i * 61) % N])
    dstp = jnp.concatenate([dst, N + (pad_i % NS)])
    srcr = srcp.reshape(NS, CHK, CH)
    dstr = dstp.reshape(NS, CHK, CH)
    iidx = jnp.stack([jnp.stack([srcr, dstr], axis=2),
                      jnp.stack([srcr + NPAD, dstr], axis=2)])
    iidx = iidx.reshape(NC * NS, CHK, 2, CH)
    dst32 = dstp.reshape(NC * NS, DCHK, CH)

    x_pad = jnp.zeros((NPAD, FEATS), jnp.float32).at[:N].set(x)

    degw = _deg_kernel(dst32)

    w1x = adj1_W[:, 1:, :].transpose(1, 0, 2).reshape(FEATS, _CH2)
    rmat = (jnp.eye(CLASSES, dtype=jnp.float32)[:, :, None]
            * adj1_W[:, 0, :][None, :, :]).reshape(CLASSES, _CH2)
    b1f = adj1_b.reshape(1, _CH2)
    w2f = adj2_W[:, :, 0].reshape(1, _CH2)
    smat = jnp.repeat(jnp.eye(CLASSES, dtype=jnp.float32), HIDDEN, axis=0)
    b2r = adj2_b[:, 0].reshape(1, CLASSES)

    u0, d2 = _prep_call(x_pad, lin1_W, lin1_b.reshape(1, HIDDEN),
                        lin2_W, lin2_b.reshape(1, CLASSES), degw)

    u20 = _diff_kernel(u0.reshape(NC * NPAD, HALF), d2, iidx)

    u0p = _adj_call(u20.reshape(NC, NPAD, HALF), degw, x_pad,
                    w1x, rmat, b1f, w2f, smat, b2r)

    u40 = _diff_kernel(u0p.reshape(NC * NPAD, HALF), d2, iidx)

    h = _fin_call(u40.reshape(NC, NPAD, HALF), degw)
    return h[:N]
